# baseline (speedup 1.0000x reference)
import functools

import jax
import jax.numpy as jnp
from jax import lax
from jax.experimental import pallas as pl
from jax.experimental.pallas import tpu as pltpu
from jax.experimental.pallas import tpu_sc as plsc

VOCAB = 100000
EMBED = 128
N_POS = 2048
N_SEG = 3
SEQ = 2048
BATCH = 4
N = SEQ * BATCH

NC = 2
NS = 16
NW = NC * NS
CHUNK = 128
ROWS_PER_W = N // NW
NCH = ROWS_PER_W // CHUNK

INV_SCALE = 1.0 / (float(EMBED) ** 0.5)
EPS = 1e-5 / float(EMBED)

ROWS_BLK = 4096


def _sc_gather2(tok_ids, comb_ids, tok_tab, comb_tab):
    @functools.partial(
        pl.kernel,
        mesh=plsc.VectorSubcoreMesh(core_axis_name="c", subcore_axis_name="s"),
        out_type=jax.ShapeDtypeStruct((N, EMBED), jnp.float32),
        scratch_types=[
            pltpu.VMEM((NCH, CHUNK), jnp.int32),
            pltpu.VMEM((NCH, CHUNK), jnp.int32),
            pltpu.VMEM((ROWS_PER_W, EMBED), jnp.float32),
            pltpu.VMEM((ROWS_PER_W, EMBED), jnp.float32),
            pltpu.SemaphoreType.DMA,
            pltpu.SemaphoreType.DMA,
        ],
    )
    def k(tok_ids_hbm, comb_ids_hbm, tok_tab_hbm, comb_tab_hbm,
          sum_out, tidx_v, cidx_v, trows_v, crows_v, gsem, wsem):
        wid = lax.axis_index("s") * NC + lax.axis_index("c")
        base = wid * ROWS_PER_W
        for c in range(NCH):
            src = pl.ds(base + c * CHUNK, CHUNK)
            pltpu.sync_copy(tok_ids_hbm.at[src], tidx_v.at[c])
            pltpu.sync_copy(comb_ids_hbm.at[src], cidx_v.at[c])
        gathers = []
        for c in range(NCH):
            dst = pl.ds(c * CHUNK, CHUNK)
            gathers.append(pltpu.async_copy(
                tok_tab_hbm.at[tidx_v.at[c]], trows_v.at[dst], gsem))
            gathers.append(pltpu.async_copy(
                comb_tab_hbm.at[cidx_v.at[c]], crows_v.at[dst], gsem))
        for d in gathers:
            d.wait()

        @plsc.parallel_loop(0, ROWS_PER_W, 1, unroll=4)
        def _row(r):
            for j in range(EMBED // 16):
                cols = pl.ds(j * 16, 16)
                trows_v[r, cols] = trows_v[r, cols] + crows_v[r, cols]

        pltpu.async_copy(trows_v, sum_out.at[pl.ds(base, ROWS_PER_W)],
                         wsem).wait()

    return k(tok_ids, comb_ids, tok_tab, comb_tab)


def _tc_body(x_ref, gam_ref, bet_ref, out_ref):
    x = x_ref[...]
    mean = jnp.mean(x, axis=1, keepdims=True)
    ctr = x - mean
    var = jnp.mean(ctr * ctr, axis=1, keepdims=True)
    out_ref[...] = ctr * lax.rsqrt(var + EPS) * gam_ref[...] + bet_ref[...]


def _tc_add_ln(x, gamma2d, beta2d):
    return pl.pallas_call(
        _tc_body,
        grid=(N // ROWS_BLK,),
        in_specs=[
            pl.BlockSpec((ROWS_BLK, EMBED), lambda i: (i, 0)),
            pl.BlockSpec((1, EMBED), lambda i: (0, 0)),
            pl.BlockSpec((1, EMBED), lambda i: (0, 0)),
        ],
        out_specs=pl.BlockSpec((ROWS_BLK, EMBED), lambda i: (i, 0)),
        out_shape=jax.ShapeDtypeStruct((N, EMBED), jnp.float32),
        compiler_params=pltpu.CompilerParams(
            dimension_semantics=("parallel",),
        ),
    )(x, gamma2d, beta2d)


def kernel(token_sequence, segment_indices, position_indices, token_table,
           segment_table, position_table, ln_gamma, ln_beta):
    tok_ids = token_sequence.astype(jnp.int32).reshape(-1)
    comb_ids = (segment_indices.astype(jnp.int32) * N_POS
                + position_indices.astype(jnp.int32)).reshape(-1)
    comb_tab = ((segment_table[:, None, :] + position_table[None, :, :])
                * INV_SCALE).reshape(N_SEG * N_POS, EMBED)
    summed = _sc_gather2(tok_ids, comb_ids, token_table, comb_tab)
    out = _tc_add_ln(summed,
                     ln_gamma.reshape(1, EMBED), ln_beta.reshape(1, EMBED))
    return out.reshape(SEQ, BATCH, EMBED)


# speedup vs baseline: 1.0525x; 1.0525x over previous
import functools

import jax
import jax.numpy as jnp
from jax import lax
from jax.experimental import pallas as pl
from jax.experimental.pallas import tpu as pltpu
from jax.experimental.pallas import tpu_sc as plsc

VOCAB = 100000
EMBED = 128
N_POS = 2048
N_SEG = 3
SEQ = 2048
BATCH = 4
N = SEQ * BATCH

NC = 2
NS = 16
NW = NC * NS
CHUNK = 128
NSLICE = 1
NS_ROWS = N // NSLICE
ROWS_PER_W = NS_ROWS // NW
NCH = ROWS_PER_W // CHUNK

INV_SCALE = 1.0 / (float(EMBED) ** 0.5)
EPS = 1e-5 / float(EMBED)

ROWS_BLK = 4096


def _sc_gather2(ids, tok_tab, comb_tab, sl):
    @functools.partial(
        pl.kernel,
        mesh=plsc.VectorSubcoreMesh(core_axis_name="c", subcore_axis_name="s"),
        out_type=[
            jax.ShapeDtypeStruct((NS_ROWS, EMBED), jnp.float32),
            jax.ShapeDtypeStruct((NS_ROWS, EMBED), jnp.float32),
        ],
        scratch_types=[
            pltpu.VMEM((NCH, CHUNK), jnp.int32),
            pltpu.VMEM((NCH, CHUNK), jnp.int32),
            pltpu.VMEM((ROWS_PER_W, EMBED), jnp.float32),
            pltpu.VMEM((ROWS_PER_W, EMBED), jnp.float32),
            pltpu.SemaphoreType.DMA,
            pltpu.SemaphoreType.DMA,
        ],
    )
    def k(ids_hbm, tok_tab_hbm, comb_tab_hbm,
          tok_out, comb_out, tidx_v, cidx_v, trows_v, crows_v, gsem, wsem):
        wid = lax.axis_index("s") * NC + lax.axis_index("c")
        base = wid * ROWS_PER_W
        pltpu.sync_copy(ids_hbm.at[0, sl, wid], tidx_v)
        pltpu.sync_copy(ids_hbm.at[1, sl, wid], cidx_v)
        gathers = []
        for c in range(NCH):
            dst = pl.ds(c * CHUNK, CHUNK)
            gathers.append(pltpu.async_copy(
                tok_tab_hbm.at[tidx_v.at[c]], trows_v.at[dst], gsem))
            gathers.append(pltpu.async_copy(
                comb_tab_hbm.at[cidx_v.at[c]], crows_v.at[dst], gsem))
        for d in gathers:
            d.wait()
        writes = [
            pltpu.async_copy(trows_v, tok_out.at[pl.ds(base, ROWS_PER_W)], wsem),
            pltpu.async_copy(crows_v, comb_out.at[pl.ds(base, ROWS_PER_W)], wsem),
        ]
        for w in writes:
            w.wait()

    return k(ids, tok_tab, comb_tab)


def _tc_body(a_ref, b_ref, gam_ref, bet_ref, out_ref):
    x = a_ref[...] + b_ref[...]
    mean = jnp.mean(x, axis=1, keepdims=True)
    ctr = x - mean
    var = jnp.mean(ctr * ctr, axis=1, keepdims=True)
    out_ref[...] = ctr * lax.rsqrt(var + EPS) * gam_ref[...] + bet_ref[...]


def _tc_add_ln(a, b, gamma2d, beta2d):
    return pl.pallas_call(
        _tc_body,
        grid=(NS_ROWS // ROWS_BLK,),
        in_specs=[
            pl.BlockSpec((ROWS_BLK, EMBED), lambda i: (i, 0)),
            pl.BlockSpec((ROWS_BLK, EMBED), lambda i: (i, 0)),
            pl.BlockSpec((1, EMBED), lambda i: (0, 0)),
            pl.BlockSpec((1, EMBED), lambda i: (0, 0)),
        ],
        out_specs=pl.BlockSpec((ROWS_BLK, EMBED), lambda i: (i, 0)),
        out_shape=jax.ShapeDtypeStruct((NS_ROWS, EMBED), jnp.float32),
        compiler_params=pltpu.CompilerParams(
            dimension_semantics=("parallel",),
        ),
    )(a, b, gamma2d, beta2d)


def kernel(token_sequence, segment_indices, position_indices, token_table,
           segment_table, position_table, ln_gamma, ln_beta):
    comb_flat = (segment_indices.astype(jnp.int32) * N_POS
                 + position_indices.astype(jnp.int32)).reshape(-1)
    ids = jnp.concatenate(
        [token_sequence.astype(jnp.int32).reshape(-1), comb_flat]
    ).reshape(2, NSLICE, NW, NCH, CHUNK)
    comb_tab = ((segment_table[:, None, :] + position_table[None, :, :])
                * INV_SCALE).reshape(N_SEG * N_POS, EMBED)
    gamma2d = ln_gamma.reshape(1, EMBED)
    beta2d = ln_beta.reshape(1, EMBED)
    tok_rows, comb_rows = _sc_gather2(ids, token_table, comb_tab, 0)
    out = _tc_add_ln(tok_rows, comb_rows, gamma2d, beta2d)
    return out.reshape(SEQ, BATCH, EMBED)
